# Initial kernel scaffold; baseline (speedup 1.0000x reference)
import jax
import jax.numpy as jnp
from jax.experimental import pallas as pl


def kernel(features, neighbors0, neighbors1, neighbors2, pools1, pools2, upsamples0, upsamples1, W_enc0, W_pool1, W_enc1, W_pool2, W_enc2, W_lat1, W_last, W_head, W_out):
    raise NotImplementedError("write your pallas kernel here")


# speedup vs baseline: 1.0909x; 1.0909x over previous
import functools

import jax
import jax.numpy as jnp
from jax import lax
from jax.experimental import pallas as pl
from jax.experimental.pallas import tpu as pltpu
import jax.experimental.pallas.tpu_sc as plsc

_K = 32
_NC, _NS = 2, 16
_NW = _NC * _NS
_L = 16

_P0, _P1, _P2 = 10240, 2560, 768


def _mesh():
    return plsc.VectorSubcoreMesh(core_axis_name="c", subcore_axis_name="s",
                                  num_cores=_NC, num_subcores=_NS)


def _wid():
    return lax.axis_index("s") * _NC + lax.axis_index("c")


def _sc_gather_sum(table2, idx3, P, CB, D):
    dc = D // 128
    R = CB * dc
    G = P // CB
    nch = G // _NW

    @functools.partial(
        pl.kernel,
        out_type=jax.ShapeDtypeStruct((P * dc, 128), jnp.float32),
        mesh=_mesh(),
        scratch_types=[
            pltpu.VMEM((_K, R), jnp.int32),
            pltpu.VMEM((R, 128), jnp.float32),
            pltpu.SemaphoreType.DMA,
        ],
    )
    def k(table_hbm, idx_hbm, out_hbm, idx_v, acc_v, sem):
        w = _wid()
        z = jnp.zeros((_L,), jnp.float32)

        def chunk(c, carry):
            g = w * nch + c
            pltpu.sync_copy(idx_hbm.at[g], idx_v)

            def zrow(i, carry2):
                for d in range(128 // _L):
                    acc_v[i, pl.ds(d * _L, _L)] = z
                return carry2

            lax.fori_loop(0, R, zrow, 0)
            descs = [
                pltpu.async_copy(table_hbm.at[idx_v.at[kk]], acc_v, sem, add=True)
                for kk in range(_K)
            ]
            for dsc in descs:
                dsc.wait()
            pltpu.sync_copy(acc_v, out_hbm.at[pl.ds(g * R, R)])
            return carry

        lax.fori_loop(0, nch, chunk, 0)

    return k(table2, idx3).reshape(P, D)


def _sc_gather_max(table, idx2, P, CB, D):
    G = P // CB
    nch = G // _NW
    M = CB * _K

    @functools.partial(
        pl.kernel,
        out_type=jax.ShapeDtypeStruct((P, D), jnp.float32),
        mesh=_mesh(),
        scratch_types=[
            pltpu.VMEM((M,), jnp.int32),
            pltpu.VMEM((M, D), jnp.float32),
            pltpu.VMEM((CB, D), jnp.float32),
            pltpu.SemaphoreType.DMA,
        ],
    )
    def k(table_hbm, idx_hbm, out_hbm, idx_v, rows_v, out_v, sem):
        w = _wid()

        def chunk(c, carry):
            g = w * nch + c
            pltpu.sync_copy(idx_hbm.at[g], idx_v)
            pltpu.async_copy(table_hbm.at[idx_v], rows_v, sem).wait()
            for p in range(CB):
                init = tuple(
                    rows_v[p * _K, pl.ds(d * _L, _L)] for d in range(D // _L)
                )

                def body(kk, m):
                    return tuple(
                        jnp.maximum(m[d], rows_v[p * _K + kk, pl.ds(d * _L, _L)])
                        for d in range(D // _L)
                    )

                m = lax.fori_loop(1, _K, body, init)
                for d in range(D // _L):
                    out_v[p, pl.ds(d * _L, _L)] = m[d]
            pltpu.sync_copy(out_v, out_hbm.at[pl.ds(g * CB, CB)])
            return carry

        lax.fori_loop(0, nch, chunk, 0)

    return k(table, idx2)


def _sc_gather_rows(table, idx2, P, CB, D):
    G = P // CB
    nch = G // _NW

    @functools.partial(
        pl.kernel,
        out_type=jax.ShapeDtypeStruct((P, D), jnp.float32),
        mesh=_mesh(),
        scratch_types=[
            pltpu.VMEM((CB,), jnp.int32),
            pltpu.VMEM((CB, D), jnp.float32),
            pltpu.SemaphoreType.DMA,
        ],
    )
    def k(table_hbm, idx_hbm, out_hbm, idx_v, rows_v, sem):
        w = _wid()

        def chunk(c, carry):
            g = w * nch + c
            pltpu.sync_copy(idx_hbm.at[g], idx_v)
            pltpu.async_copy(table_hbm.at[idx_v], rows_v, sem).wait()
            pltpu.sync_copy(rows_v, out_hbm.at[pl.ds(g * CB, CB)])
            return carry

        lax.fori_loop(0, nch, chunk, 0)

    return k(table, idx2)


def _tc_mm(x, W, scale=None, relu=True, br=512):
    N, Di = x.shape
    Do = W.shape[1]

    def body(x_ref, w_ref, o_ref):
        xb = x_ref[...]
        if scale is not None:
            xb = xb * scale
        y = jnp.dot(xb, w_ref[...], preferred_element_type=jnp.float32)
        if relu:
            y = jnp.maximum(y, 0.0)
        o_ref[...] = y

    return pl.pallas_call(
        body,
        grid=(N // br,),
        in_specs=[
            pl.BlockSpec((br, Di), lambda i: (i, 0)),
            pl.BlockSpec((Di, Do), lambda i: (0, 0)),
        ],
        out_specs=pl.BlockSpec((br, Do), lambda i: (i, 0)),
        out_shape=jax.ShapeDtypeStruct((N, Do), jnp.float32),
    )(x, W)


def _tc_enc2_lat(s2, W_enc2, Wl1b):
    N, D = s2.shape
    Do = Wl1b.shape[1]

    def body(s_ref, we_ref, wb_ref, x2_ref, z2_ref):
        x2 = jnp.maximum(
            jnp.dot(s_ref[...] * (1.0 / _K), we_ref[...],
                    preferred_element_type=jnp.float32), 0.0)
        x2_ref[...] = x2
        z2_ref[...] = jnp.dot(x2, wb_ref[...], preferred_element_type=jnp.float32)

    return pl.pallas_call(
        body,
        out_shape=(
            jax.ShapeDtypeStruct((N, D), jnp.float32),
            jax.ShapeDtypeStruct((N, Do), jnp.float32),
        ),
    )(s2, W_enc2, Wl1b)


def _tc_lat1(x1, u1, Wl1a, Wlb, br=512):
    N, D = x1.shape
    Do = Wlb.shape[1]

    def body(x_ref, u_ref, wa_ref, wb_ref, o_ref):
        h = jnp.maximum(
            jnp.dot(x_ref[...], wa_ref[...], preferred_element_type=jnp.float32)
            + u_ref[...], 0.0)
        o_ref[...] = jnp.dot(h, wb_ref[...], preferred_element_type=jnp.float32)

    return pl.pallas_call(
        body,
        grid=(N // br,),
        in_specs=[
            pl.BlockSpec((br, D), lambda i: (i, 0)),
            pl.BlockSpec((br, D), lambda i: (i, 0)),
            pl.BlockSpec((D, D), lambda i: (0, 0)),
            pl.BlockSpec((D, Do), lambda i: (0, 0)),
        ],
        out_specs=pl.BlockSpec((br, Do), lambda i: (i, 0)),
        out_shape=jax.ShapeDtypeStruct((N, Do), jnp.float32),
    )(x1, u1, Wl1a, Wlb)


def _tc_head(x0, u0, Wla, W_head, W_out, br=1024):
    N, D = x0.shape
    C = W_out.shape[1]

    def body(x_ref, u_ref, wa_ref, wh_ref, wo_ref, o_ref):
        t = jnp.maximum(
            jnp.dot(x_ref[...], wa_ref[...], preferred_element_type=jnp.float32)
            + u_ref[...], 0.0)
        t = jnp.maximum(
            jnp.dot(t, wh_ref[...], preferred_element_type=jnp.float32), 0.0)
        o_ref[...] = jnp.dot(t, wo_ref[...], preferred_element_type=jnp.float32)

    return pl.pallas_call(
        body,
        grid=(N // br,),
        in_specs=[
            pl.BlockSpec((br, D), lambda i: (i, 0)),
            pl.BlockSpec((br, D), lambda i: (i, 0)),
            pl.BlockSpec((D, D), lambda i: (0, 0)),
            pl.BlockSpec((D, D), lambda i: (0, 0)),
            pl.BlockSpec((D, C), lambda i: (0, 0)),
        ],
        out_specs=pl.BlockSpec((br, C), lambda i: (i, 0)),
        out_shape=jax.ShapeDtypeStruct((N, C), jnp.float32),
    )(x0, u0, Wla, W_head, W_out)


def _pack_neigh(n, P, CB, dc):
    G = P // CB
    n = jnp.pad(n, ((0, P - n.shape[0]), (0, 0))).astype(jnp.int32)
    q = n.reshape(G, CB, _K) * dc
    q = q[:, :, :, None] + jnp.arange(dc, dtype=jnp.int32)
    return q.transpose(0, 2, 1, 3).reshape(G, _K, CB * dc)


def _pack_pool(p, P, CB):
    G = P // CB
    p = jnp.pad(p, ((0, P - p.shape[0]), (0, 0)))
    return p.reshape(G, CB * _K).astype(jnp.int32)


def _pack_ups(u, P, CB):
    G = P // CB
    u = jnp.pad(u, (0, P - u.shape[0]))
    return u.reshape(G, CB).astype(jnp.int32)


def kernel(features, neighbors0, neighbors1, neighbors2, pools1, pools2,
           upsamples0, upsamples1,
           W_enc0, W_pool1, W_enc1, W_pool2, W_enc2, W_lat1, W_last, W_head,
           W_out):
    D0, D1, D2 = 128, 256, 512
    N0 = features.shape[0]

    n0 = _pack_neigh(neighbors0, _P0, 64, 1)
    n1 = _pack_neigh(neighbors1, _P1, 40, 2)
    n2 = _pack_neigh(neighbors2, _P2, 24, 4)
    p1 = _pack_pool(pools1, _P1, 4)
    p2 = _pack_pool(pools2, _P2, 4)
    u0 = _pack_ups(upsamples0, _P0, 64)
    u1 = _pack_ups(upsamples1, _P1, 80)

    Wl1a, Wl1b = W_lat1[:D1], W_lat1[D1:]
    Wla, Wlb = W_last[:D0], W_last[D0:]

    s0 = _sc_gather_sum(features, n0, _P0, 64, D0)
    x0 = _tc_mm(s0, W_enc0, scale=1.0 / _K, br=1024)
    m1 = _sc_gather_max(x0, p1, _P1, 4, D0)
    h1 = _tc_mm(m1, W_pool1, br=512)
    s1 = _sc_gather_sum(h1.reshape(_P1 * 2, 128), n1, _P1, 40, D1)
    x1 = _tc_mm(s1, W_enc1, scale=1.0 / _K, br=512)
    m2 = _sc_gather_max(x1, p2, _P2, 4, D1)
    h2 = _tc_mm(m2, W_pool2, br=768)
    s2 = _sc_gather_sum(h2.reshape(_P2 * 4, 128), n2, _P2, 24, D2)
    x2, z2 = _tc_enc2_lat(s2, W_enc2, Wl1b)

    uu1 = _sc_gather_rows(z2, u1, _P1, 80, D1)
    z1 = _tc_lat1(x1, uu1, Wl1a, Wlb, br=512)
    uu0 = _sc_gather_rows(z1, u0, _P0, 64, D0)
    logits = _tc_head(x0, uu0, Wla, W_head, W_out, br=1024)

    return logits[:N0]
